# baseline (speedup 1.0000x reference)
import functools

import jax
import jax.numpy as jnp
from jax import lax
from jax.experimental import pallas as pl
from jax.experimental.pallas import tpu as pltpu
from jax.experimental.pallas import tpu_sc as plsc

_NC = 2
_NS = 16
_NW = _NC * _NS
_L = 16


def kernel(x, perm):
    B, T, N, C = x.shape
    nb = N // 128
    cblk = C // 8
    units = (B * T * cblk) // _NW
    hb = nb // 2

    V = x.reshape(B, T, nb, 128, cblk, 8).transpose(0, 1, 4, 2, 5, 3)
    perm3 = jnp.asarray(perm, jnp.int32).reshape(T, nb, 128)

    mesh = plsc.VectorSubcoreMesh(core_axis_name="c", subcore_axis_name="s")

    @functools.partial(
        pl.kernel,
        mesh=mesh,
        out_type=jax.ShapeDtypeStruct((B, T, cblk, nb, 8, 128), x.dtype),
        compiler_params=pltpu.CompilerParams(needs_layout_passes=False),
        scratch_types=[
            pltpu.VMEM((nb, 128), jnp.int32),
            pltpu.VMEM((nb, 128), jnp.int32),
            pltpu.VMEM((nb, 8, 128), jnp.float32),
            pltpu.VMEM((nb, 8, 128), jnp.float32),
            pltpu.VMEM((hb, 8, 128), jnp.float32),
            pltpu.VMEM((hb, 8, 128), jnp.float32),
            pltpu.SemaphoreType.DMA,
            pltpu.SemaphoreType.DMA,
            pltpu.SemaphoreType.DMA,
            pltpu.SemaphoreType.DMA,
        ],
    )
    def scramble(v_hbm, perm_hbm, o_hbm,
                 pbuf0, pbuf1, sbuf0, sbuf1, obufa, obufb,
                 sem_i, sem_p, sem_oa, sem_ob):
        wid = lax.axis_index("s") * _NC + lax.axis_index("c")

        def coords(u):
            return u // (T * cblk), (u // cblk) % T, u % cblk

        b0, t0, c0 = coords(wid)
        pltpu.async_copy(v_hbm.at[b0, t0, c0], sbuf0, sem_i)
        pltpu.async_copy(perm_hbm.at[t0], pbuf0, sem_p)

        @pl.loop(0, units // 2)
        def _(g):
            for e, pbuf, sbuf, nx_pbuf, nx_sbuf in (
                    (0, pbuf0, sbuf0, pbuf1, sbuf1),
                    (1, pbuf1, sbuf1, pbuf0, sbuf0)):
                ui = g * 2 + e
                unit = ui * _NW + wid
                b, t, cb = coords(unit)

                pltpu.make_async_copy(v_hbm.at[0, 0, 0], sbuf, sem_i).wait()
                pltpu.make_async_copy(perm_hbm.at[0], pbuf, sem_p).wait()

                @pl.when(ui < units - 1)
                def _():
                    bn, tn, cn = coords(unit + _NW)
                    pltpu.async_copy(v_hbm.at[bn, tn, cn], nx_sbuf, sem_i)
                    pltpu.async_copy(perm_hbm.at[tn], nx_pbuf, sem_p)

                for half, obuf, sem_o in ((0, obufa, sem_oa),
                                          (1, obufb, sem_ob)):
                    @pl.when(ui > 0)
                    def _():
                        pltpu.make_async_copy(
                            o_hbm.at[0, 0, 0].at[pl.ds(0, hb)], obuf,
                            sem_o).wait()

                    @plsc.parallel_loop(0, hb, unroll=8)
                    def _(vj):
                        vb = half * hb + vj
                        for k in range(128 // _L):
                            pv = pbuf[vb, pl.ds(k * _L, _L)]
                            lo = pv & 127
                            hi = (pv >> 7) * 8
                            for sl in range(8):
                                obuf[vj, sl, pl.ds(k * _L, _L)] = (
                                    plsc.load_gather(
                                        sbuf.reshape((nb * 8, 128)),
                                        [hi + sl, lo]))

                    pltpu.async_copy(
                        obuf, o_hbm.at[b, t, cb].at[pl.ds(half * hb, hb)],
                        sem_o)

        pltpu.make_async_copy(
            o_hbm.at[0, 0, 0].at[pl.ds(0, hb)], obufa, sem_oa).wait()
        pltpu.make_async_copy(
            o_hbm.at[0, 0, 0].at[pl.ds(0, hb)], obufb, sem_ob).wait()

    out6 = scramble(V, perm3)
    return out6.transpose(0, 1, 3, 5, 2, 4).reshape(B, T, N, C)


# speedup vs baseline: 1.2131x; 1.2131x over previous
import functools

import jax
import jax.numpy as jnp
from jax import lax
from jax.experimental import pallas as pl
from jax.experimental.pallas import tpu as pltpu
from jax.experimental.pallas import tpu_sc as plsc

_NC = 2
_NS = 16
_NW = _NC * _NS
_L = 16


def kernel(x, perm):
    B, T, N, C = x.shape
    nb = N // 128
    cblk = C // 8
    units = (B * T * cblk) // _NW
    hb = nb // 2

    V = x.reshape(B, T, nb, 128, cblk, 8).transpose(0, 1, 4, 2, 5, 3)
    perm3 = jnp.asarray(perm, jnp.int32).reshape(T, nb, 128)

    mesh = plsc.VectorSubcoreMesh(core_axis_name="c", subcore_axis_name="s")

    @functools.partial(
        pl.kernel,
        mesh=mesh,
        out_type=jax.ShapeDtypeStruct((B, T, cblk, nb, 8, 128), x.dtype),
        compiler_params=pltpu.CompilerParams(needs_layout_passes=False),
        scratch_types=[
            pltpu.VMEM((nb, 128), jnp.int32),
            pltpu.VMEM((nb, 128), jnp.int32),
            pltpu.VMEM((nb, 8, 128), jnp.float32),
            pltpu.VMEM((nb, 8, 128), jnp.float32),
            pltpu.VMEM((hb, 8, 128), jnp.float32),
            pltpu.VMEM((hb, 8, 128), jnp.float32),
            pltpu.SemaphoreType.DMA,
            pltpu.SemaphoreType.DMA,
            pltpu.SemaphoreType.DMA,
            pltpu.SemaphoreType.DMA,
        ],
    )
    def scramble(v_hbm, perm_hbm, o_hbm,
                 pbuf0, pbuf1, sbuf0, sbuf1, obufa, obufb,
                 sem_i, sem_p, sem_oa, sem_ob):
        wid = lax.axis_index("s") * _NC + lax.axis_index("c")
        w4 = wid // cblk
        cb = wid % cblk

        pltpu.async_copy(perm_hbm.at[w4], pbuf0, sem_p)
        pltpu.async_copy(perm_hbm.at[w4 + T // 2], pbuf1, sem_p)
        pltpu.async_copy(v_hbm.at[0, w4, cb], sbuf0, sem_i)
        pltpu.make_async_copy(perm_hbm.at[0], pbuf0, sem_p).wait()
        pltpu.make_async_copy(perm_hbm.at[0], pbuf1, sem_p).wait()

        @pl.loop(0, units // 2)
        def _(g):
            for e, pbuf, sbuf, nx_sbuf in (
                    (0, pbuf0, sbuf0, sbuf1),
                    (1, pbuf1, sbuf1, sbuf0)):
                ui = g * 2 + e
                bt = ui * (T // 2) + w4
                b = bt // T
                t = bt % T

                pltpu.make_async_copy(v_hbm.at[0, 0, 0], sbuf, sem_i).wait()

                @pl.when(ui < units - 1)
                def _():
                    btn = bt + T // 2
                    pltpu.async_copy(
                        v_hbm.at[btn // T, btn % T, cb], nx_sbuf, sem_i)

                for half, obuf, sem_o in ((0, obufa, sem_oa),
                                          (1, obufb, sem_ob)):
                    @pl.when(ui > 0)
                    def _():
                        pltpu.make_async_copy(
                            o_hbm.at[0, 0, 0].at[pl.ds(0, hb)], obuf,
                            sem_o).wait()

                    @plsc.parallel_loop(0, hb, unroll=4)
                    def _(vj):
                        vb = half * hb + vj
                        for k in range(128 // _L):
                            pv = pbuf[vb, pl.ds(k * _L, _L)]
                            lo = pv & 127
                            hi = (pv >> 7) * 8
                            for sl in range(8):
                                obuf[vj, sl, pl.ds(k * _L, _L)] = (
                                    plsc.load_gather(
                                        sbuf.reshape((nb * 8, 128)),
                                        [hi + sl, lo]))

                    pltpu.async_copy(
                        obuf, o_hbm.at[b, t, cb].at[pl.ds(half * hb, hb)],
                        sem_o)

        pltpu.make_async_copy(
            o_hbm.at[0, 0, 0].at[pl.ds(0, hb)], obufa, sem_oa).wait()
        pltpu.make_async_copy(
            o_hbm.at[0, 0, 0].at[pl.ds(0, hb)], obufb, sem_ob).wait()

    out6 = scramble(V, perm3)
    return out6.transpose(0, 1, 3, 5, 2, 4).reshape(B, T, N, C)
